# baseline (speedup 1.0000x reference)
import jax
import jax.numpy as jnp
from jax.experimental import pallas as pl
from jax.experimental.pallas import tpu as pltpu


def _gcn_body(adj_ref, nodes_ref, w_ref, out_ref):
    a = adj_ref[...].astype(jnp.bfloat16)
    x = nodes_ref[...].astype(jnp.bfloat16)
    agg = jnp.dot(a, x, preferred_element_type=jnp.float32)
    w = w_ref[...].astype(jnp.bfloat16)
    out_ref[...] = jnp.dot(agg.astype(jnp.bfloat16), w,
                           preferred_element_type=jnp.float32)


def kernel(adjacency, nodes, weights):
    adjacency = adjacency.astype(jnp.float32)
    nodes = nodes.astype(jnp.float32)
    weights = weights.astype(jnp.float32)

    B, W, N, _ = adjacency.shape
    Fin = nodes.shape[-1]
    Wp, _, Fout = weights.shape
    w_used = weights[Wp - W:, :, :]

    return pl.pallas_call(
        _gcn_body,
        grid=(W, B),
        in_specs=[
            pl.BlockSpec((None, None, N, N), lambda w, b: (b, w, 0, 0)),
            pl.BlockSpec((None, None, N, Fin), lambda w, b: (b, w, 0, 0)),
            pl.BlockSpec((None, Fin, Fout), lambda w, b: (w, 0, 0)),
        ],
        out_specs=pl.BlockSpec((None, None, N, Fout), lambda w, b: (b, w, 0, 0)),
        out_shape=jax.ShapeDtypeStruct((B, W, N, Fout), jnp.float32),
        compiler_params=pltpu.CompilerParams(
            dimension_semantics=("parallel", "parallel")),
    )(adjacency, nodes, w_used)


# speedup vs baseline: 2.8564x; 2.8564x over previous
import jax
import jax.numpy as jnp
from jax.experimental import pallas as pl
from jax.experimental.pallas import tpu as pltpu


def _gcn_body(adj_ref, nodes_ref, w_ref, out_ref):
    a = adj_ref[...].astype(jnp.bfloat16)
    x = nodes_ref[...].astype(jnp.bfloat16)
    agg = jax.lax.dot_general(
        a, x, (((2,), (1,)), ((0,), (0,))),
        preferred_element_type=jnp.float32).astype(jnp.bfloat16)
    w = w_ref[...].astype(jnp.bfloat16)
    out_ref[...] = jax.lax.dot_general(
        agg, w, (((2,), (1,)), ((0,), (0,))),
        preferred_element_type=jnp.float32)


def kernel(adjacency, nodes, weights):
    adjacency = adjacency.astype(jnp.float32)
    nodes = nodes.astype(jnp.float32)
    weights = weights.astype(jnp.float32)

    B, W, N, _ = adjacency.shape
    Fin = nodes.shape[-1]
    Wp, _, Fout = weights.shape
    w_used = weights[Wp - W:, :, :]

    bn = N // 2 if N % 2 == 0 and N >= 256 else N
    nb = N // bn

    return pl.pallas_call(
        _gcn_body,
        grid=(B, nb),
        in_specs=[
            pl.BlockSpec((None, W, bn, N), lambda b, n: (b, 0, n, 0)),
            pl.BlockSpec((None, W, N, Fin), lambda b, n: (b, 0, 0, 0)),
            pl.BlockSpec((W, Fin, Fout), lambda b, n: (0, 0, 0)),
        ],
        out_specs=pl.BlockSpec((None, W, bn, Fout), lambda b, n: (b, 0, n, 0)),
        out_shape=jax.ShapeDtypeStruct((B, W, N, Fout), jnp.float32),
        compiler_params=pltpu.CompilerParams(
            dimension_semantics=("parallel", "parallel")),
    )(adjacency, nodes, w_used)


# speedup vs baseline: 3.6617x; 1.2819x over previous
import jax
import jax.numpy as jnp
from jax.experimental import pallas as pl
from jax.experimental.pallas import tpu as pltpu


def _gcn_body(adj_ref, nodes_ref, w_ref, out_ref):
    a = adj_ref[...].astype(jnp.bfloat16)
    x = nodes_ref[...].astype(jnp.bfloat16)
    agg = jax.lax.dot_general(
        a, x, (((2,), (1,)), ((0,), (0,))),
        preferred_element_type=jnp.float32).astype(jnp.bfloat16)
    w = w_ref[...].astype(jnp.bfloat16)
    out_ref[...] = jax.lax.dot_general(
        agg, w, (((2,), (1,)), ((0,), (0,))),
        preferred_element_type=jnp.float32)


def kernel(adjacency, nodes, weights):
    adjacency = adjacency.astype(jnp.float32)
    nodes = nodes.astype(jnp.float32)
    weights = weights.astype(jnp.float32)

    B, W, N, _ = adjacency.shape
    Fin = nodes.shape[-1]
    Wp, _, Fout = weights.shape
    w_used = weights[Wp - W:, :, :]

    bn = N
    nb = N // bn

    return pl.pallas_call(
        _gcn_body,
        grid=(B, nb),
        in_specs=[
            pl.BlockSpec((None, W, bn, N), lambda b, n: (b, 0, n, 0)),
            pl.BlockSpec((None, W, N, Fin), lambda b, n: (b, 0, 0, 0)),
            pl.BlockSpec((W, Fin, Fout), lambda b, n: (0, 0, 0)),
        ],
        out_specs=pl.BlockSpec((None, W, bn, Fout), lambda b, n: (b, 0, n, 0)),
        out_shape=jax.ShapeDtypeStruct((B, W, N, Fout), jnp.float32),
        compiler_params=pltpu.CompilerParams(
            dimension_semantics=("parallel", "parallel")),
    )(adjacency, nodes, w_used)
